# baseline (speedup 1.0000x reference)
import functools

import jax
import jax.numpy as jnp
from jax import lax
from jax.experimental import pallas as pl
from jax.experimental.pallas import tpu as pltpu
from jax.experimental.pallas import tpu_sc as plsc

_EMB_DIM = 64


@functools.lru_cache(maxsize=None)
def _build(R: int, T: int, D: int):
    info = plsc.get_sparse_core_info()
    NC, NS = info.num_cores, info.num_subcores
    NW = NC * NS
    assert R % NW == 0
    RW = R // NW
    CX = 32
    NCH = RW // CX
    CT = CX * T
    b_per_w = RW * T
    assert NCH * CX == RW

    mesh = plsc.VectorSubcoreMesh(core_axis_name="c", subcore_axis_name="s")

    @functools.partial(
        pl.kernel,
        out_type=jax.ShapeDtypeStruct((R, T, D), jnp.float32),
        mesh=mesh,
        scratch_types=[
            pltpu.VMEM((b_per_w,), jnp.int32),
            pltpu.VMEM((CT, D), jnp.float32),
            pltpu.VMEM((CT, D), jnp.float32),
            pltpu.SemaphoreType.DMA,
            pltpu.SemaphoreType.DMA,
            pltpu.SemaphoreType.DMA,
            pltpu.SemaphoreType.DMA,
        ],
        compiler_params=pltpu.CompilerParams(use_tc_tiling_on_sc=False),
    )
    def emb(table_hbm, idx_hbm, out_hbm, idx_v, rows0, rows1, g0, g1, w0, w1):
        wid = lax.axis_index("s") * NC + lax.axis_index("c")
        rbase = wid * RW
        bufs, gsems, wsems = [rows0, rows1], [g0, g1], [w0, w1]
        pltpu.sync_copy(idx_hbm.at[pl.ds(rbase * T, b_per_w)], idx_v)

        def issue_gather(i):
            b = i % 2
            return pltpu.async_copy(
                table_hbm.at[idx_v.at[pl.ds(i * CT, CT)]],
                bufs[b],
                gsems[b],
            )

        def issue_writebacks(i):
            b = i % 2
            return [
                pltpu.async_copy(
                    bufs[b].at[pl.ds(j * T, T)],
                    out_hbm.at[rbase + i * CX + j],
                    wsems[b],
                )
                for j in range(CX)
            ]

        ghandle = issue_gather(0)
        whandles = [None] * NCH
        for i in range(NCH):
            b = i % 2
            ghandle.wait()
            whandles[i] = issue_writebacks(i)
            if i + 1 < NCH:
                if i >= 1:
                    for h in whandles[i - 1]:
                        h.wait()
                ghandle = issue_gather(i + 1)
        for h in whandles[NCH - 2]:
            h.wait()
        for h in whandles[NCH - 1]:
            h.wait()

    return emb


def kernel(x, emb_weight):
    emb = _build(x.shape[0], x.shape[1], _EMB_DIM)
    flat_idx = x.reshape(-1).astype(jnp.int32)
    return emb(emb_weight, flat_idx)
